# baseline (speedup 1.0000x reference)
import jax
import jax.numpy as jnp
from jax.experimental import pallas as pl
from jax.experimental.pallas import tpu as pltpu

B = 16384
A = 1000
NCHUNK = 8
CR = B // NCHUNK


def _dma_copy_kernel(act_ref, q_ref, sav_ref, out_ref, *sems):
    del act_ref, q_ref
    for c in range(NCHUNK):
        pltpu.make_async_copy(
            sav_ref.at[pl.ds(c * CR, CR), :],
            out_ref.at[pl.ds(c * CR, CR), :],
            sems[c],
        ).start()
    for c in range(NCHUNK):
        pltpu.make_async_copy(
            sav_ref.at[pl.ds(c * CR, CR), :],
            out_ref.at[pl.ds(c * CR, CR), :],
            sems[c],
        ).wait()


def kernel(state_action_values, action, q_prime):
    act = action[:, 0].astype(jnp.int32)
    return pl.pallas_call(
        _dma_copy_kernel,
        in_specs=[
            pl.BlockSpec(memory_space=pl.ANY),
            pl.BlockSpec(memory_space=pl.ANY),
            pl.BlockSpec(memory_space=pl.ANY),
        ],
        out_specs=pl.BlockSpec(memory_space=pl.ANY),
        out_shape=jax.ShapeDtypeStruct((B, A), jnp.float32),
        scratch_shapes=[pltpu.SemaphoreType.DMA] * NCHUNK,
    )(act, q_prime, state_action_values)


# speedup vs baseline: 13.4811x; 13.4811x over previous
import jax
import jax.numpy as jnp
from jax.experimental import pallas as pl
from jax.experimental.pallas import tpu as pltpu

B = 16384
A = 1000
BR = 2048


def _scatter_copy_kernel(act_ref, q_ref, sav_ref, out_ref):
    act = act_ref[:]
    q = q_ref[:]
    col = jax.lax.broadcasted_iota(jnp.int32, (BR, A), 1)
    mask = col == act[:, None]
    out_ref[...] = jnp.where(mask, q[:, None], sav_ref[...])


def kernel(state_action_values, action, q_prime):
    act = action[:, 0].astype(jnp.int32)
    grid = (B // BR,)
    return pl.pallas_call(
        _scatter_copy_kernel,
        grid=grid,
        in_specs=[
            pl.BlockSpec((BR,), lambda i: (i,)),
            pl.BlockSpec((BR,), lambda i: (i,)),
            pl.BlockSpec((BR, A), lambda i: (i, 0)),
        ],
        out_specs=pl.BlockSpec((BR, A), lambda i: (i, 0)),
        out_shape=jax.ShapeDtypeStruct((B, A), jnp.float32),
        compiler_params=pltpu.CompilerParams(
            dimension_semantics=("parallel",)),
    )(act, q_prime, state_action_values)
